# baseline (speedup 1.0000x reference)
import functools

import jax
import jax.numpy as jnp
from jax import lax
from jax.experimental import pallas as pl
from jax.experimental.pallas import tpu as pltpu
from jax.experimental.pallas import tpu_sc as plsc

N = 10000
E = 160000
D = 256
H = 128
NC = 2
NS = 16
C = 128

SP_ROWS = 10240
DUMMY = N

E_PAD = 163840
EPT3 = E_PAD // (NC * NS)
G = 8
GROUPS = E_PAD // (NS * C * G)


@functools.cache
def _make_sc_conv():
    mesh = plsc.VectorSubcoreMesh(core_axis_name="c", subcore_axis_name="s",
                                  num_cores=NC, num_subcores=NS)

    @functools.partial(
        pl.kernel, mesh=mesh,
        out_type=jax.ShapeDtypeStruct((NC * SP_ROWS, H), jnp.float32),
        scratch_types=[
            pltpu.VMEM_SHARED((SP_ROWS, H), jnp.float32),
            pltpu.VMEM((C, H), jnp.float32),
            pltpu.VMEM((C, H), jnp.float32),
            pltpu.VMEM((2 * G, C), jnp.int32),
            pltpu.VMEM((2 * G, 1, C), jnp.int32),
            pltpu.SemaphoreType.DMA,
            pltpu.SemaphoreType.DMA,
            pltpu.SemaphoreType.DMA,
            pltpu.SemaphoreType.DMA,
            pltpu.SemaphoreType.DMA,
        ],
    )
    def conv(src2, dst3, ta, tb, zrows, agg_out,
             agg_s, rows0, rows1, sidx, didx, gsem, ssem0, ssem1,
             isem, isem2):
        cid = lax.axis_index("c")
        sid = lax.axis_index("s")
        rows = (rows0, rows1)
        ssem = (ssem0, ssem1)

        pltpu.sync_copy(zrows, rows0)
        for k in range(SP_ROWS // (NS * C)):
            r0 = sid * (SP_ROWS // NS) + k * C
            pltpu.sync_copy(rows0, agg_s.at[pl.ds(r0, C)])
        plsc.subcore_barrier()

        def run_edges(table):
            def idx_copies(g, slot):
                rb = sid * (GROUPS * G) + g * G
                return (pltpu.make_async_copy(
                            src2.at[pl.ds(rb, G)],
                            sidx.at[pl.ds(slot * G, G)], isem),
                        pltpu.make_async_copy(
                            dst3.at[pl.ds(rb, G)],
                            didx.at[pl.ds(slot * G, G)], isem2))

            for d in idx_copies(0, 0):
                d.start()

            def group(g, _):
                o = (g % 2) * G
                for d in idx_copies(g, g % 2):
                    d.wait()

                @pl.when(g + 1 < GROUPS)
                def _():
                    for d in idx_copies(g + 1, (g + 1) % 2):
                        d.start()

                gd = pltpu.async_copy(table.at[sidx.at[o]], rows0, gsem)
                sds = [None, None]
                for j in range(G):
                    b = j % 2
                    gd.wait()
                    sds[b] = pltpu.async_copy(
                        rows[b], agg_s.at[didx.at[o + j, 0]], ssem[b],
                        add=True)
                    if j + 1 < G:
                        nb = (j + 1) % 2
                        if sds[nb] is not None:
                            sds[nb].wait()
                        gd = pltpu.async_copy(table.at[sidx.at[o + j + 1]],
                                              rows[nb], gsem)
                sds[0].wait()
                sds[1].wait()
                return 0

            lax.fori_loop(0, GROUPS, group, 0)

        @pl.when(cid == 0)
        def _():
            run_edges(ta)

        @pl.when(cid == 1)
        def _():
            run_edges(tb)

        plsc.subcore_barrier()

        for k in range(SP_ROWS // (NS * C)):
            r0 = sid * (SP_ROWS // NS) + k * C
            pltpu.sync_copy(agg_s.at[pl.ds(r0, C)],
                            agg_out.at[pl.ds(cid * SP_ROWS + r0, C)])

    return conv


@functools.cache
def _make_sc_deg():
    mesh = plsc.VectorSubcoreMesh(core_axis_name="c", subcore_axis_name="s",
                                  num_cores=NC, num_subcores=NS)

    @functools.partial(
        pl.kernel, mesh=mesh,
        out_type=jax.ShapeDtypeStruct((NC * SP_ROWS, H), jnp.float32),
        scratch_types=[
            pltpu.VMEM_SHARED((SP_ROWS, H), jnp.float32),
            pltpu.VMEM((C, H), jnp.float32),
            pltpu.VMEM((C, H), jnp.float32),
            pltpu.VMEM((G, 1, C), jnp.int32),
            pltpu.SemaphoreType.DMA,
        ],
    )
    def deg(dst3, zrows, orows, deg_out, deg_s, o128, buf, didx, ssem):
        cid = lax.axis_index("c")
        sid = lax.axis_index("s")
        pltpu.sync_copy(orows, o128)
        pltpu.sync_copy(zrows, buf)
        for k in range(SP_ROWS // (NS * C)):
            r0 = sid * (SP_ROWS // NS) + k * C
            pltpu.sync_copy(buf, deg_s.at[pl.ds(r0, C)])
        plsc.subcore_barrier()

        ept = E_PAD // (NC * NS)

        def group(g, _):
            rb = (cid * NS + sid) * (ept // C) + g * G
            pltpu.sync_copy(dst3.at[pl.ds(rb, G)], didx)
            sds = [pltpu.async_copy(o128, deg_s.at[didx.at[j, 0]], ssem,
                                    add=True) for j in range(G)]
            for dsc in sds:
                dsc.wait()
            return 0

        lax.fori_loop(0, ept // (C * G), group, 0)
        plsc.subcore_barrier()

        for k in range(SP_ROWS // (NS * C)):
            r0 = sid * (SP_ROWS // NS) + k * C
            pltpu.sync_copy(deg_s.at[pl.ds(r0, C)], buf)
            pltpu.sync_copy(buf, deg_out.at[pl.ds(cid * SP_ROWS + r0, C)])

    return deg


def _tc1_body(agg, dega, degb, x, w1n, w1r, b1, out, outa, outb):
    scale = 1.0 / jnp.maximum(dega[:] + degb[:], 1.0)
    h = (jnp.dot(agg[:] * scale, w1n[:], preferred_element_type=jnp.float32)
         + jnp.dot(x[:], w1r[:], preferred_element_type=jnp.float32)
         + b1[:])
    h = jnp.maximum(h, 0.0)
    out[:] = h
    outa[:] = h[:, :H]
    outb[:] = h[:, H:]


def _tc2_body(agg, dega, degb, h1, w2n, w2r, b2, g, b,
              wda, wdb, bd, h_out, s_out, t_out):
    scale = 1.0 / jnp.maximum(dega[:] + degb[:], 1.0)
    h2 = (jnp.dot(agg[:] * scale, w2n[:], preferred_element_type=jnp.float32)
          + jnp.dot(h1[:], w2r[:], preferred_element_type=jnp.float32)
          + b2[:])
    h2 = jnp.maximum(h2, 0.0)
    mu = jnp.mean(h2, axis=-1, keepdims=True)
    var = jnp.mean((h2 - mu) ** 2, axis=-1, keepdims=True)
    hn = (h2 - mu) * jax.lax.rsqrt(var + 1e-5) * g[:] + b[:]
    h_out[:] = hn
    s_out[:] = jnp.dot(hn, wda[:], preferred_element_type=jnp.float32) + bd[:]
    t_out[:] = jnp.dot(hn, wdb[:], preferred_element_type=jnp.float32)


@functools.cache
def _make_sc_decode():
    mesh = plsc.VectorSubcoreMesh(core_axis_name="c", subcore_axis_name="s",
                                  num_cores=NC, num_subcores=NS)

    @functools.partial(
        pl.kernel, mesh=mesh,
        out_type=jax.ShapeDtypeStruct((E_PAD,), jnp.float32),
        compiler_params=pltpu.CompilerParams(needs_layout_passes=False),
        scratch_types=[
            pltpu.VMEM((SP_ROWS,), jnp.float32),
            pltpu.VMEM((SP_ROWS,), jnp.float32),
            pltpu.VMEM((EPT3,), jnp.int32),
            pltpu.VMEM((EPT3,), jnp.int32),
            pltpu.VMEM((EPT3,), jnp.float32),
        ],
    )
    def decode(s_hbm, t_hbm, src, dst, out, sv, tv, si, di, ov):
        cid = lax.axis_index("c")
        sid = lax.axis_index("s")
        wid = cid * NS + sid
        base = wid * EPT3
        pltpu.sync_copy(s_hbm, sv)
        pltpu.sync_copy(t_hbm, tv)
        pltpu.sync_copy(src.at[pl.ds(base, EPT3)], si)
        pltpu.sync_copy(dst.at[pl.ds(base, EPT3)], di)

        def step(i, _):
            sl = pl.ds(i * 16, 16)
            a = plsc.load_gather(sv, [si[sl]])
            c = plsc.load_gather(tv, [di[sl]])
            ov[sl] = 1.0 / (1.0 + jnp.exp(-(a + c)))
            return 0

        lax.fori_loop(0, EPT3 // 16, step, 0)
        pltpu.sync_copy(ov, out.at[pl.ds(base, EPT3)])

    return decode


def kernel(x, edge_index, W1n, W1r, b1, W2n, W2r, b2, ln_g, ln_b, Wd, bd):
    src = edge_index[0]
    dst = edge_index[1]
    pad = E_PAD - E
    srcp = jnp.concatenate([src, jnp.zeros((pad,), jnp.int32)])
    dstp = jnp.concatenate([dst, jnp.full((pad,), DUMMY, jnp.int32)])
    srcp2 = srcp.reshape(E_PAD // C, C)
    dst3d = dstp.reshape(E_PAD // C, 1, C)

    zrows = jnp.zeros((C, H), jnp.float32)
    orows = jnp.ones((C, H), jnp.float32)

    degc = _make_sc_deg()(dst3d, zrows, orows)
    dega = degc[:N, :1]
    degb = degc[SP_ROWS:SP_ROWS + N, :1]
    agg1c = _make_sc_conv()(srcp2, dst3d, x[:, :H], x[:, H:], zrows)
    agg1 = jnp.concatenate([agg1c[:N], agg1c[SP_ROWS:SP_ROWS + N]], axis=1)

    B = 2000
    NB = N // B
    row = pl.BlockSpec((B, D), lambda i: (i, 0))
    half = pl.BlockSpec((B, H), lambda i: (i, 0))
    col1 = pl.BlockSpec((B, 1), lambda i: (i, 0))
    wfull = pl.BlockSpec((D, D), lambda i: (0, 0))
    vfull = pl.BlockSpec((D,), lambda i: (0,))

    h1, h1a, h1b = pl.pallas_call(
        _tc1_body,
        grid=(NB,),
        in_specs=[row, col1, col1, row, wfull, wfull, vfull],
        out_specs=[row, half, half],
        out_shape=[
            jax.ShapeDtypeStruct((N, D), jnp.float32),
            jax.ShapeDtypeStruct((N, H), jnp.float32),
            jax.ShapeDtypeStruct((N, H), jnp.float32),
        ],
    )(agg1, dega, degb, x, W1n, W1r, b1)

    agg2c = _make_sc_conv()(srcp2, dst3d, h1a, h1b, zrows)
    agg2 = jnp.concatenate([agg2c[:N], agg2c[SP_ROWS:SP_ROWS + N]], axis=1)

    wcol = pl.BlockSpec((D, 1), lambda i: (0, 0))
    h, s, t = pl.pallas_call(
        _tc2_body,
        grid=(NB,),
        in_specs=[row, col1, col1, row, wfull, wfull, vfull,
                  vfull, vfull, wcol, wcol, pl.BlockSpec((1,), lambda i: (0,))],
        out_specs=[row, col1, col1],
        out_shape=[
            jax.ShapeDtypeStruct((N, D), jnp.float32),
            jax.ShapeDtypeStruct((N, 1), jnp.float32),
            jax.ShapeDtypeStruct((N, 1), jnp.float32),
        ],
    )(agg2, dega, degb, h1, W2n, W2r, b2, ln_g, ln_b, Wd[:D], Wd[D:], bd)

    sp = jnp.pad(s.reshape(N), (0, SP_ROWS - N))
    tp = jnp.pad(t.reshape(N), (0, SP_ROWS - N))
    newp = _make_sc_decode()(sp, tp, srcp, dstp)
    new_h = newp[:E, None]
    return (h, new_h)
